# baseline (speedup 1.0000x reference)
import functools

import jax
import jax.numpy as jnp
from jax import lax
from jax.experimental import pallas as pl
from jax.experimental.pallas import tpu as pltpu
from jax.experimental.pallas import tpu_sc as plsc

CH = 128


def _make_gather(n, e, h):
    mesh = plsc.VectorSubcoreMesh(core_axis_name="c", subcore_axis_name="s")
    nw = mesh.num_cores * mesh.num_subcores
    nchunks = e // CH
    full_rounds = nchunks // nw
    rem = nchunks - full_rounds * nw

    @functools.partial(
        pl.kernel,
        out_type=jax.ShapeDtypeStruct((e, h), jnp.float32),
        mesh=mesh,
        scratch_types=[
            pltpu.VMEM((CH,), jnp.int32),
            pltpu.VMEM((CH, h), jnp.float32),
            pltpu.SemaphoreType.DMA,
        ],
        compiler_params=pltpu.CompilerParams(use_tc_tiling_on_sc=False),
    )
    def gather_k(x_hbm, src_hbm, out_hbm, idx_v, rows_v, sem):
        wid = lax.axis_index("s") * mesh.num_cores + lax.axis_index("c")

        def do_chunk(cid):
            off = pl.multiple_of(cid * CH, CH)
            pltpu.sync_copy(src_hbm.at[pl.ds(off, CH)], idx_v)
            pltpu.async_copy(x_hbm.at[idx_v], rows_v, sem).wait()
            pltpu.sync_copy(rows_v, out_hbm.at[pl.ds(off, CH)])

        def body(g, carry):
            do_chunk(g * nw + wid)
            return carry

        lax.fori_loop(0, full_rounds, body, 0)
        if rem:
            @pl.when(wid < rem)
            def _():
                do_chunk(full_rounds * nw + wid)

    return gather_k


def _make_scatter(n, e, h):
    mesh = plsc.VectorSubcoreMesh(core_axis_name="c", subcore_axis_name="s")
    nc, ns = mesh.num_cores, mesh.num_subcores
    nw = nc * ns
    nchunks = e // CH
    full_rounds = nchunks // nw
    rem = nchunks - full_rounds * nw
    rows_per_sub = n // ns

    @functools.partial(
        pl.kernel,
        out_type=jax.ShapeDtypeStruct((nc, n, h), jnp.float32),
        mesh=mesh,
        scratch_types=[
            pltpu.VMEM((CH,), jnp.int32),
            pltpu.VMEM((CH, h), jnp.float32),
            pltpu.VMEM_SHARED((n, h), jnp.float32),
            pltpu.SemaphoreType.DMA,
        ],
        compiler_params=pltpu.CompilerParams(use_tc_tiling_on_sc=False),
    )
    def scatter_k(vals_hbm, dst_hbm, zeros_hbm, out_hbm, idx_v, rows_v,
                  acc_sh, sem):
        cid_ax = lax.axis_index("c")
        sid = lax.axis_index("s")
        wid = sid * nc + cid_ax

        @pl.when(sid == 0)
        def _():
            pltpu.sync_copy(zeros_hbm, acc_sh)

        plsc.subcore_barrier()

        def do_chunk(cid):
            off = pl.multiple_of(cid * CH, CH)
            pltpu.sync_copy(dst_hbm.at[pl.ds(off, CH)], idx_v)
            pltpu.sync_copy(vals_hbm.at[pl.ds(off, CH)], rows_v)
            pltpu.sync_copy(rows_v, acc_sh.at[idx_v], add=True)

        def body(g, carry):
            do_chunk(g * nw + wid)
            return carry

        lax.fori_loop(0, full_rounds, body, 0)
        if rem:
            @pl.when(wid < rem)
            def _():
                do_chunk(full_rounds * nw + wid)

        plsc.subcore_barrier()
        r0 = sid * rows_per_sub
        pltpu.sync_copy(acc_sh.at[pl.ds(r0, rows_per_sub)],
                        out_hbm.at[cid_ax, pl.ds(r0, rows_per_sub)])

    return scatter_k


def _make_msg_body(eb, h):
    def _msg_body(xj_ref, ea_ref, w1t_ref, b1_ref, m2_ref, b2r_ref, o_ref):
        xj = xj_ref[...]
        hid = jnp.maximum(ea_ref[...] @ w1t_ref[...] + b1_ref[...], 0.0)
        a = lax.broadcast_in_dim(xj, (eb, h, h), (0, 1)).reshape(eb, h * h)
        b = jnp.tile(hid, (1, h))
        o_ref[...] = (a * b) @ m2_ref[...] + xj @ b2r_ref[...]
    return _msg_body


def _make_msg(e, h, ed, eb):
    grid = e // eb
    full = lambda i: (0, 0)
    return pl.pallas_call(
        _make_msg_body(eb, h),
        grid=(grid,),
        in_specs=[
            pl.BlockSpec((eb, h), lambda i: (i, 0)),
            pl.BlockSpec((eb, ed), lambda i: (i, 0)),
            pl.BlockSpec((ed, h), full),
            pl.BlockSpec((1, h), full),
            pl.BlockSpec((h * h, h), full),
            pl.BlockSpec((h, h), full),
        ],
        out_specs=pl.BlockSpec((eb, h), lambda i: (i, 0)),
        out_shape=jax.ShapeDtypeStruct((e, h), jnp.float32),
    )


def _gru_body(p0_ref, p1_ref, c0_ref, c1_ref, x_ref, root_ref, bias_ref,
              wr_ref, wz_ref, wn_ref, ur_ref, uz_ref, un_ref,
              bir_ref, biz_ref, bin_ref, bhr_ref, bhz_ref, bhn_ref, o_ref):
    x = x_ref[...]
    cnt = c0_ref[...] + c1_ref[...]
    denom = jnp.maximum(cnt, 1.0)
    agg = (p0_ref[...] + p1_ref[...]) / denom
    conv = agg + x @ root_ref[...] + bias_ref[...]
    m = jnp.maximum(conv, 0.0)
    r = jax.nn.sigmoid(m @ wr_ref[...] + bir_ref[...]
                       + x @ ur_ref[...] + bhr_ref[...])
    z = jax.nn.sigmoid(m @ wz_ref[...] + biz_ref[...]
                       + x @ uz_ref[...] + bhz_ref[...])
    nwe = jnp.tanh(m @ wn_ref[...] + bin_ref[...]
                   + r * (x @ un_ref[...] + bhn_ref[...]))
    o_ref[...] = (1.0 - z) * nwe + z * x


def _make_gru(n, h):
    specs = ([pl.BlockSpec((n, h))] * 4
             + [pl.BlockSpec((n, h))]
             + [pl.BlockSpec((h, h)), pl.BlockSpec((1, h))]
             + [pl.BlockSpec((h, h))] * 6
             + [pl.BlockSpec((1, h))] * 6)
    return pl.pallas_call(
        _gru_body,
        in_specs=specs,
        out_specs=pl.BlockSpec((n, h)),
        out_shape=jax.ShapeDtypeStruct((n, h), jnp.float32),
    )


def kernel(out, edge_index, edge_attr, W1, b1, W2, b2, root, bias,
           w_ih, w_hh, b_ih, b_hh):
    n, h = out.shape
    e, ed = edge_attr.shape
    src = edge_index[0]
    dst = edge_index[1]

    w1t = W1.T
    b1r = b1.reshape(1, h)
    w2r3 = W2.reshape(h, h, h)
    m2 = w2r3.transpose(0, 2, 1).reshape(h * h, h)
    b2r = b2.reshape(h, h)
    wr, wz, wn = (w_ih[0:h].T, w_ih[h:2 * h].T, w_ih[2 * h:3 * h].T)
    ur, uz, un = (w_hh[0:h].T, w_hh[h:2 * h].T, w_hh[2 * h:3 * h].T)
    bir, biz, bin_ = (b_ih[0:h].reshape(1, h), b_ih[h:2 * h].reshape(1, h),
                      b_ih[2 * h:3 * h].reshape(1, h))
    bhr, bhz, bhn = (b_hh[0:h].reshape(1, h), b_hh[h:2 * h].reshape(1, h),
                     b_hh[2 * h:3 * h].reshape(1, h))
    biasr = bias.reshape(1, h)
    zeros = jnp.zeros((n, h), jnp.float32)
    ones = jnp.ones((e, h), jnp.float32)

    gather_fn = _make_gather(n, e, h)
    scatter_fn = _make_scatter(n, e, h)
    msg_fn = _make_msg(e, h, ed, eb=1000)
    gru_fn = _make_gru(n, h)

    cntp = scatter_fn(ones, dst, zeros)
    x = out
    for _ in range(3):
        x_j = gather_fn(x, src)
        msg = msg_fn(x_j, edge_attr, w1t, b1r, m2, b2r)
        aggp = scatter_fn(msg, dst, zeros)
        x = gru_fn(aggp[0], aggp[1], cntp[0], cntp[1], x, root, biasr,
                   wr, wz, wn, ur, uz, un, bir, biz, bin_, bhr, bhz, bhn)
    return x


# speedup vs baseline: 2.5676x; 2.5676x over previous
import functools

import jax
import jax.numpy as jnp
from jax import lax
from jax.experimental import pallas as pl
from jax.experimental.pallas import tpu as pltpu
from jax.experimental.pallas import tpu_sc as plsc

CH = 128


def _make_gather(n, e, h):
    mesh = plsc.VectorSubcoreMesh(core_axis_name="c", subcore_axis_name="s")
    nw = mesh.num_cores * mesh.num_subcores
    nchunks = e // CH
    full_rounds = nchunks // nw
    rem = nchunks - full_rounds * nw

    @functools.partial(
        pl.kernel,
        out_type=jax.ShapeDtypeStruct((e, h), jnp.float32),
        mesh=mesh,
        scratch_types=[
            pltpu.VMEM((CH,), jnp.int32),
            pltpu.VMEM((CH, h), jnp.float32),
            pltpu.SemaphoreType.DMA,
        ],
        compiler_params=pltpu.CompilerParams(use_tc_tiling_on_sc=False),
    )
    def gather_k(x_hbm, src_hbm, out_hbm, idx_v, rows_v, sem):
        wid = lax.axis_index("s") * mesh.num_cores + lax.axis_index("c")

        def do_chunk(cid):
            off = pl.multiple_of(cid * CH, CH)
            pltpu.sync_copy(src_hbm.at[pl.ds(off, CH)], idx_v)
            pltpu.async_copy(x_hbm.at[idx_v], rows_v, sem).wait()
            pltpu.sync_copy(rows_v, out_hbm.at[pl.ds(off, CH)])

        def body(g, carry):
            do_chunk(g * nw + wid)
            return carry

        lax.fori_loop(0, full_rounds, body, 0)
        if rem:
            @pl.when(wid < rem)
            def _():
                do_chunk(full_rounds * nw + wid)

    return gather_k


def _make_scatter(n, e, h):
    mesh = plsc.VectorSubcoreMesh(core_axis_name="c", subcore_axis_name="s")
    nc, ns = mesh.num_cores, mesh.num_subcores
    nw = nc * ns
    nchunks = e // CH
    full_rounds = nchunks // nw
    rem = nchunks - full_rounds * nw
    rows_per_sub = n // ns

    @functools.partial(
        pl.kernel,
        out_type=jax.ShapeDtypeStruct((nc, n, h), jnp.float32),
        mesh=mesh,
        scratch_types=[
            pltpu.VMEM((CH,), jnp.int32),
            pltpu.VMEM((CH, h), jnp.float32),
            pltpu.VMEM_SHARED((n, h), jnp.float32),
            pltpu.SemaphoreType.DMA,
        ],
        compiler_params=pltpu.CompilerParams(use_tc_tiling_on_sc=False),
    )
    def scatter_k(vals_hbm, dst_hbm, zeros_hbm, out_hbm, idx_v, rows_v,
                  acc_sh, sem):
        cid_ax = lax.axis_index("c")
        sid = lax.axis_index("s")
        wid = sid * nc + cid_ax

        @pl.when(sid == 0)
        def _():
            pltpu.sync_copy(zeros_hbm, acc_sh)

        plsc.subcore_barrier()

        def do_chunk(cid):
            off = pl.multiple_of(cid * CH, CH)
            pltpu.sync_copy(dst_hbm.at[pl.ds(off, CH)], idx_v)
            pltpu.sync_copy(vals_hbm.at[pl.ds(off, CH)], rows_v)
            pltpu.sync_copy(rows_v, acc_sh.at[idx_v], add=True)

        def body(g, carry):
            do_chunk(g * nw + wid)
            return carry

        lax.fori_loop(0, full_rounds, body, 0)
        if rem:
            @pl.when(wid < rem)
            def _():
                do_chunk(full_rounds * nw + wid)

        plsc.subcore_barrier()
        r0 = sid * rows_per_sub
        pltpu.sync_copy(acc_sh.at[pl.ds(r0, rows_per_sub)],
                        out_hbm.at[cid_ax, pl.ds(r0, rows_per_sub)])

    return scatter_k


def _msg_body(xj_ref, ea_ref, w1t_ref, b1_ref, e1_ref, e2_ref, m2_ref,
              b2r_ref, o_ref):
    xj = xj_ref[...]
    hid = jnp.maximum(ea_ref[...] @ w1t_ref[...] + b1_ref[...], 0.0)
    xb = xj.astype(jnp.bfloat16)
    hb = hid.astype(jnp.bfloat16)
    a = lax.dot(xb, e1_ref[...], preferred_element_type=jnp.float32)
    b = lax.dot(hb, e2_ref[...], preferred_element_type=jnp.float32)
    z = (a * b).astype(jnp.bfloat16)
    o_ref[...] = (lax.dot(z, m2_ref[...], preferred_element_type=jnp.float32)
                  + xj @ b2r_ref[...])


def _make_msg(e, h, ed, eb):
    grid = e // eb
    full = lambda i: (0, 0)
    return pl.pallas_call(
        _msg_body,
        grid=(grid,),
        in_specs=[
            pl.BlockSpec((eb, h), lambda i: (i, 0)),
            pl.BlockSpec((eb, ed), lambda i: (i, 0)),
            pl.BlockSpec((ed, h), full),
            pl.BlockSpec((1, h), full),
            pl.BlockSpec((h, h * h), full),
            pl.BlockSpec((h, h * h), full),
            pl.BlockSpec((h * h, h), full),
            pl.BlockSpec((h, h), full),
        ],
        out_specs=pl.BlockSpec((eb, h), lambda i: (i, 0)),
        out_shape=jax.ShapeDtypeStruct((e, h), jnp.float32),
    )


def _gru_body(p0_ref, p1_ref, c0_ref, c1_ref, x_ref, root_ref, bias_ref,
              wr_ref, wz_ref, wn_ref, ur_ref, uz_ref, un_ref,
              bir_ref, biz_ref, bin_ref, bhr_ref, bhz_ref, bhn_ref, o_ref):
    x = x_ref[...]
    cnt = c0_ref[...] + c1_ref[...]
    denom = jnp.maximum(cnt, 1.0)
    agg = (p0_ref[...] + p1_ref[...]) / denom
    conv = agg + x @ root_ref[...] + bias_ref[...]
    m = jnp.maximum(conv, 0.0)
    r = jax.nn.sigmoid(m @ wr_ref[...] + bir_ref[...]
                       + x @ ur_ref[...] + bhr_ref[...])
    z = jax.nn.sigmoid(m @ wz_ref[...] + biz_ref[...]
                       + x @ uz_ref[...] + bhz_ref[...])
    nwe = jnp.tanh(m @ wn_ref[...] + bin_ref[...]
                   + r * (x @ un_ref[...] + bhn_ref[...]))
    o_ref[...] = (1.0 - z) * nwe + z * x


def _make_gru(n, h):
    specs = ([pl.BlockSpec((n, h))] * 4
             + [pl.BlockSpec((n, h))]
             + [pl.BlockSpec((h, h)), pl.BlockSpec((1, h))]
             + [pl.BlockSpec((h, h))] * 6
             + [pl.BlockSpec((1, h))] * 6)
    return pl.pallas_call(
        _gru_body,
        in_specs=specs,
        out_specs=pl.BlockSpec((n, h)),
        out_shape=jax.ShapeDtypeStruct((n, h), jnp.float32),
    )


def kernel(out, edge_index, edge_attr, W1, b1, W2, b2, root, bias,
           w_ih, w_hh, b_ih, b_hh):
    n, h = out.shape
    e, ed = edge_attr.shape
    src = edge_index[0]
    dst = edge_index[1]

    w1t = W1.T
    b1r = b1.reshape(1, h)
    w2r3 = W2.reshape(h, h, h)
    m2 = w2r3.transpose(0, 2, 1).reshape(h * h, h)
    b2r = b2.reshape(h, h)
    eye = jnp.eye(h, dtype=jnp.bfloat16)
    e1 = jnp.kron(eye, jnp.ones((1, h), jnp.bfloat16))
    e2 = jnp.kron(jnp.ones((1, h), jnp.bfloat16), eye)
    m2b = m2.astype(jnp.bfloat16)
    wr, wz, wn = (w_ih[0:h].T, w_ih[h:2 * h].T, w_ih[2 * h:3 * h].T)
    ur, uz, un = (w_hh[0:h].T, w_hh[h:2 * h].T, w_hh[2 * h:3 * h].T)
    bir, biz, bin_ = (b_ih[0:h].reshape(1, h), b_ih[h:2 * h].reshape(1, h),
                      b_ih[2 * h:3 * h].reshape(1, h))
    bhr, bhz, bhn = (b_hh[0:h].reshape(1, h), b_hh[h:2 * h].reshape(1, h),
                     b_hh[2 * h:3 * h].reshape(1, h))
    biasr = bias.reshape(1, h)
    zeros = jnp.zeros((n, h), jnp.float32)
    ones = jnp.ones((e, h), jnp.float32)

    gather_fn = _make_gather(n, e, h)
    scatter_fn = _make_scatter(n, e, h)
    msg_fn = _make_msg(e, h, ed, eb=1000)
    gru_fn = _make_gru(n, h)

    cntp = scatter_fn(ones, dst, zeros)
    x = out
    for _ in range(3):
        x_j = gather_fn(x, src)
        msg = msg_fn(x_j, edge_attr, w1t, b1r, e1, e2, m2b, b2r)
        aggp = scatter_fn(msg, dst, zeros)
        x = gru_fn(aggp[0], aggp[1], cntp[0], cntp[1], x, root, biasr,
                   wr, wz, wn, ur, uz, un, bir, biz, bin_, bhr, bhz, bhn)
    return x


# speedup vs baseline: 2.8799x; 1.1216x over previous
import functools

import jax
import jax.numpy as jnp
from jax import lax
from jax.experimental import pallas as pl
from jax.experimental.pallas import tpu as pltpu
from jax.experimental.pallas import tpu_sc as plsc

CH = 128
K = 5


def _make_gather(n, e, h):
    mesh = plsc.VectorSubcoreMesh(core_axis_name="c", subcore_axis_name="s")
    nw = mesh.num_cores * mesh.num_subcores
    ew = e // nw
    full = ew // CH
    tail = ew - full * CH
    supers = full // K
    rem_chunks = full - supers * K

    @functools.partial(
        pl.kernel,
        out_type=jax.ShapeDtypeStruct((e, h), jnp.float32),
        mesh=mesh,
        scratch_types=[
            [pltpu.VMEM((CH,), jnp.int32) for _ in range(K)],
            [pltpu.VMEM((CH, h), jnp.float32) for _ in range(K)],
            pltpu.VMEM((8,), jnp.int32),
            pltpu.VMEM((8, h), jnp.float32),
            pltpu.SemaphoreType.DMA,
            pltpu.SemaphoreType.DMA,
            pltpu.SemaphoreType.DMA,
        ],
        compiler_params=pltpu.CompilerParams(use_tc_tiling_on_sc=False),
    )
    def gather_k(x_hbm, src_hbm, out_hbm, idx_vs, rows_vs, idx_t, rows_t,
                 sem_i, sem_g, sem_s):
        wid = lax.axis_index("s") * mesh.num_cores + lax.axis_index("c")
        base = pl.multiple_of(wid * ew, 8)

        def run_group(first_cid, cnt):
            di = [pltpu.async_copy(
                      src_hbm.at[pl.ds(base + (first_cid + j) * CH, CH)],
                      idx_vs[j], sem_i)
                  for j in range(cnt)]
            dg = []
            for j in range(cnt):
                di[j].wait()
                dg.append(pltpu.async_copy(x_hbm.at[idx_vs[j]], rows_vs[j],
                                           sem_g))
            ds_ = []
            for j in range(cnt):
                dg[j].wait()
                ds_.append(pltpu.async_copy(
                    rows_vs[j],
                    out_hbm.at[pl.ds(base + (first_cid + j) * CH, CH)],
                    sem_s))
            for d in ds_:
                d.wait()

        def body(g, carry):
            run_group(g * K, K)
            return carry

        lax.fori_loop(0, supers, body, 0)
        if rem_chunks:
            run_group(supers * K, rem_chunks)
        if tail:
            toff = pl.multiple_of(base + full * CH, 8)
            pltpu.sync_copy(src_hbm.at[pl.ds(toff, tail)], idx_t)
            pltpu.async_copy(x_hbm.at[idx_t], rows_t, sem_g).wait()
            pltpu.sync_copy(rows_t, out_hbm.at[pl.ds(toff, tail)])

    return gather_k


def _make_scatter(n, e, h):
    mesh = plsc.VectorSubcoreMesh(core_axis_name="c", subcore_axis_name="s")
    nc, ns = mesh.num_cores, mesh.num_subcores
    nw = nc * ns
    ew = e // nw
    full = ew // CH
    tail = ew - full * CH
    supers = full // K
    rem_chunks = full - supers * K
    rows_per_sub = n // ns

    @functools.partial(
        pl.kernel,
        out_type=jax.ShapeDtypeStruct((nc, n, h), jnp.float32),
        mesh=mesh,
        scratch_types=[
            [pltpu.VMEM((CH,), jnp.int32) for _ in range(K)],
            [pltpu.VMEM((CH, h), jnp.float32) for _ in range(K)],
            pltpu.VMEM((8,), jnp.int32),
            pltpu.VMEM((8, h), jnp.float32),
            pltpu.VMEM_SHARED((n, h), jnp.float32),
            pltpu.SemaphoreType.DMA,
            pltpu.SemaphoreType.DMA,
            pltpu.SemaphoreType.DMA,
        ],
        compiler_params=pltpu.CompilerParams(use_tc_tiling_on_sc=False),
    )
    def scatter_k(vals_hbm, dst_hbm, zeros_hbm, out_hbm, idx_vs, rows_vs,
                  idx_t, rows_t, acc_sh, sem_i, sem_v, sem_a):
        cid_ax = lax.axis_index("c")
        sid = lax.axis_index("s")
        wid = sid * nc + cid_ax
        base = pl.multiple_of(wid * ew, 8)

        @pl.when(sid == 0)
        def _():
            pltpu.sync_copy(zeros_hbm, acc_sh)

        plsc.subcore_barrier()

        def run_group(first_cid, cnt):
            di, dv = [], []
            for j in range(cnt):
                off = base + (first_cid + j) * CH
                di.append(pltpu.async_copy(dst_hbm.at[pl.ds(off, CH)],
                                           idx_vs[j], sem_i))
                dv.append(pltpu.async_copy(vals_hbm.at[pl.ds(off, CH)],
                                           rows_vs[j], sem_v))
            da = []
            for j in range(cnt):
                di[j].wait()
                dv[j].wait()
                da.append(pltpu.async_copy(rows_vs[j], acc_sh.at[idx_vs[j]],
                                           sem_a, add=True))
            for d in da:
                d.wait()

        def body(g, carry):
            run_group(g * K, K)
            return carry

        lax.fori_loop(0, supers, body, 0)
        if rem_chunks:
            run_group(supers * K, rem_chunks)
        if tail:
            toff = pl.multiple_of(base + full * CH, 8)
            pltpu.sync_copy(dst_hbm.at[pl.ds(toff, tail)], idx_t)
            pltpu.sync_copy(vals_hbm.at[pl.ds(toff, tail)], rows_t)
            pltpu.sync_copy(rows_t, acc_sh.at[idx_t], add=True)

        plsc.subcore_barrier()
        r0 = sid * rows_per_sub
        pltpu.sync_copy(acc_sh.at[pl.ds(r0, rows_per_sub)],
                        out_hbm.at[cid_ax, pl.ds(r0, rows_per_sub)])

    return scatter_k


def _msg_body(xj_ref, ea_ref, w1t_ref, b1_ref, e1_ref, e2_ref, m2_ref,
              b2r_ref, o_ref):
    xj = xj_ref[...]
    hid = jnp.maximum(ea_ref[...] @ w1t_ref[...] + b1_ref[...], 0.0)
    xb = xj.astype(jnp.bfloat16)
    hb = hid.astype(jnp.bfloat16)
    a = lax.dot(xb, e1_ref[...], preferred_element_type=jnp.float32)
    b = lax.dot(hb, e2_ref[...], preferred_element_type=jnp.float32)
    z = (a * b).astype(jnp.bfloat16)
    o_ref[...] = (lax.dot(z, m2_ref[...], preferred_element_type=jnp.float32)
                  + xj @ b2r_ref[...])


def _make_msg(e, h, ed, eb):
    grid = e // eb
    full = lambda i: (0, 0)
    return pl.pallas_call(
        _msg_body,
        grid=(grid,),
        in_specs=[
            pl.BlockSpec((eb, h), lambda i: (i, 0)),
            pl.BlockSpec((eb, ed), lambda i: (i, 0)),
            pl.BlockSpec((ed, h), full),
            pl.BlockSpec((1, h), full),
            pl.BlockSpec((h, h * h), full),
            pl.BlockSpec((h, h * h), full),
            pl.BlockSpec((h * h, h), full),
            pl.BlockSpec((h, h), full),
        ],
        out_specs=pl.BlockSpec((eb, h), lambda i: (i, 0)),
        out_shape=jax.ShapeDtypeStruct((e, h), jnp.float32),
    )


def _gru_body(p0_ref, p1_ref, c0_ref, c1_ref, x_ref, root_ref, bias_ref,
              wr_ref, wz_ref, wn_ref, ur_ref, uz_ref, un_ref,
              bir_ref, biz_ref, bin_ref, bhr_ref, bhz_ref, bhn_ref, o_ref):
    x = x_ref[...]
    cnt = c0_ref[...] + c1_ref[...]
    denom = jnp.maximum(cnt, 1.0)
    agg = (p0_ref[...] + p1_ref[...]) / denom
    conv = agg + x @ root_ref[...] + bias_ref[...]
    m = jnp.maximum(conv, 0.0)
    r = jax.nn.sigmoid(m @ wr_ref[...] + bir_ref[...]
                       + x @ ur_ref[...] + bhr_ref[...])
    z = jax.nn.sigmoid(m @ wz_ref[...] + biz_ref[...]
                       + x @ uz_ref[...] + bhz_ref[...])
    nwe = jnp.tanh(m @ wn_ref[...] + bin_ref[...]
                   + r * (x @ un_ref[...] + bhn_ref[...]))
    o_ref[...] = (1.0 - z) * nwe + z * x


def _make_gru(n, h):
    specs = ([pl.BlockSpec((n, h))] * 4
             + [pl.BlockSpec((n, h))]
             + [pl.BlockSpec((h, h)), pl.BlockSpec((1, h))]
             + [pl.BlockSpec((h, h))] * 6
             + [pl.BlockSpec((1, h))] * 6)
    return pl.pallas_call(
        _gru_body,
        in_specs=specs,
        out_specs=pl.BlockSpec((n, h)),
        out_shape=jax.ShapeDtypeStruct((n, h), jnp.float32),
    )


def kernel(out, edge_index, edge_attr, W1, b1, W2, b2, root, bias,
           w_ih, w_hh, b_ih, b_hh):
    n, h = out.shape
    e, ed = edge_attr.shape
    src = edge_index[0]
    dst = edge_index[1]

    w1t = W1.T
    b1r = b1.reshape(1, h)
    w2r3 = W2.reshape(h, h, h)
    m2 = w2r3.transpose(0, 2, 1).reshape(h * h, h)
    b2r = b2.reshape(h, h)
    eye = jnp.eye(h, dtype=jnp.bfloat16)
    e1 = jnp.kron(eye, jnp.ones((1, h), jnp.bfloat16))
    e2 = jnp.kron(jnp.ones((1, h), jnp.bfloat16), eye)
    m2b = m2.astype(jnp.bfloat16)
    wr, wz, wn = (w_ih[0:h].T, w_ih[h:2 * h].T, w_ih[2 * h:3 * h].T)
    ur, uz, un = (w_hh[0:h].T, w_hh[h:2 * h].T, w_hh[2 * h:3 * h].T)
    bir, biz, bin_ = (b_ih[0:h].reshape(1, h), b_ih[h:2 * h].reshape(1, h),
                      b_ih[2 * h:3 * h].reshape(1, h))
    bhr, bhz, bhn = (b_hh[0:h].reshape(1, h), b_hh[h:2 * h].reshape(1, h),
                     b_hh[2 * h:3 * h].reshape(1, h))
    biasr = bias.reshape(1, h)
    zeros = jnp.zeros((n, h), jnp.float32)
    ones = jnp.ones((e, h), jnp.float32)

    gather_fn = _make_gather(n, e, h)
    scatter_fn = _make_scatter(n, e, h)
    msg_fn = _make_msg(e, h, ed, eb=1000)
    gru_fn = _make_gru(n, h)

    cntp = scatter_fn(ones, dst, zeros)
    x = out
    for _ in range(3):
        x_j = gather_fn(x, src)
        msg = msg_fn(x_j, edge_attr, w1t, b1r, e1, e2, m2b, b2r)
        aggp = scatter_fn(msg, dst, zeros)
        x = gru_fn(aggp[0], aggp[1], cntp[0], cntp[1], x, root, biasr,
                   wr, wz, wn, ur, uz, un, bir, biz, bin_, bhr, bhz, bhn)
    return x


# speedup vs baseline: 3.9109x; 1.3580x over previous
import functools

import jax
import jax.numpy as jnp
from jax import lax
from jax.experimental import pallas as pl
from jax.experimental.pallas import tpu as pltpu
from jax.experimental.pallas import tpu_sc as plsc

CH = 128
K = 5


def _make_gather(n, e, h):
    mesh = plsc.VectorSubcoreMesh(core_axis_name="c", subcore_axis_name="s")
    nw = mesh.num_cores * mesh.num_subcores
    ew = e // nw
    full = ew // CH
    tail = ew - full * CH
    supers = full // K
    rem_chunks = full - supers * K

    @functools.partial(
        pl.kernel,
        out_type=jax.ShapeDtypeStruct((e, h), jnp.float32),
        mesh=mesh,
        scratch_types=[
            [pltpu.VMEM((CH,), jnp.int32) for _ in range(K)],
            [pltpu.VMEM((CH, h), jnp.float32) for _ in range(K)],
            pltpu.VMEM((8,), jnp.int32),
            pltpu.VMEM((8, h), jnp.float32),
            pltpu.SemaphoreType.DMA,
            pltpu.SemaphoreType.DMA,
            pltpu.SemaphoreType.DMA,
        ],
        compiler_params=pltpu.CompilerParams(use_tc_tiling_on_sc=False),
    )
    def gather_k(x_hbm, src_hbm, out_hbm, idx_vs, rows_vs, idx_t, rows_t,
                 sem_i, sem_g, sem_s):
        wid = lax.axis_index("s") * mesh.num_cores + lax.axis_index("c")
        base = pl.multiple_of(wid * ew, 8)

        def run_group(first_cid, cnt):
            di = [pltpu.async_copy(
                      src_hbm.at[pl.ds(base + (first_cid + j) * CH, CH)],
                      idx_vs[j], sem_i)
                  for j in range(cnt)]
            dg = []
            for j in range(cnt):
                di[j].wait()
                dg.append(pltpu.async_copy(x_hbm.at[idx_vs[j]], rows_vs[j],
                                           sem_g))
            ds_ = []
            for j in range(cnt):
                dg[j].wait()
                ds_.append(pltpu.async_copy(
                    rows_vs[j],
                    out_hbm.at[pl.ds(base + (first_cid + j) * CH, CH)],
                    sem_s))
            for d in ds_:
                d.wait()

        def body(g, carry):
            run_group(g * K, K)
            return carry

        lax.fori_loop(0, supers, body, 0)
        if rem_chunks:
            run_group(supers * K, rem_chunks)
        if tail:
            toff = pl.multiple_of(base + full * CH, 8)
            pltpu.sync_copy(src_hbm.at[pl.ds(toff, tail)], idx_t)
            pltpu.async_copy(x_hbm.at[idx_t], rows_t, sem_g).wait()
            pltpu.sync_copy(rows_t, out_hbm.at[pl.ds(toff, tail)])

    return gather_k


def _make_scatter(n, e, h):
    mesh = plsc.VectorSubcoreMesh(core_axis_name="c", subcore_axis_name="s")
    nc, ns = mesh.num_cores, mesh.num_subcores
    nw = nc * ns
    ew = e // nw
    full = ew // CH
    tail = ew - full * CH
    supers = full // K
    rem_chunks = full - supers * K
    rows_per_sub = n // ns

    @functools.partial(
        pl.kernel,
        out_type=jax.ShapeDtypeStruct((nc, n, h), jnp.float32),
        mesh=mesh,
        scratch_types=[
            [pltpu.VMEM((CH,), jnp.int32) for _ in range(K)],
            [pltpu.VMEM((CH, h), jnp.float32) for _ in range(K)],
            pltpu.VMEM((8,), jnp.int32),
            pltpu.VMEM((8, h), jnp.float32),
            pltpu.VMEM_SHARED((n, h), jnp.float32),
            pltpu.SemaphoreType.DMA,
            pltpu.SemaphoreType.DMA,
            pltpu.SemaphoreType.DMA,
        ],
        compiler_params=pltpu.CompilerParams(use_tc_tiling_on_sc=False),
    )
    def scatter_k(vals_hbm, dst_hbm, zeros_hbm, out_hbm, idx_vs, rows_vs,
                  idx_t, rows_t, acc_sh, sem_i, sem_v, sem_a):
        cid_ax = lax.axis_index("c")
        sid = lax.axis_index("s")
        wid = sid * nc + cid_ax
        base = pl.multiple_of(wid * ew, 8)

        @pl.when(sid == 0)
        def _():
            pltpu.sync_copy(zeros_hbm, acc_sh)

        plsc.subcore_barrier()

        def run_group(first_cid, cnt):
            di, dv = [], []
            for j in range(cnt):
                off = base + (first_cid + j) * CH
                di.append(pltpu.async_copy(dst_hbm.at[pl.ds(off, CH)],
                                           idx_vs[j], sem_i))
                dv.append(pltpu.async_copy(vals_hbm.at[pl.ds(off, CH)],
                                           rows_vs[j], sem_v))
            da = []
            for j in range(cnt):
                di[j].wait()
                dv[j].wait()
                da.append(pltpu.async_copy(rows_vs[j], acc_sh.at[idx_vs[j]],
                                           sem_a, add=True))
            for d in da:
                d.wait()

        def body(g, carry):
            run_group(g * K, K)
            return carry

        lax.fori_loop(0, supers, body, 0)
        if rem_chunks:
            run_group(supers * K, rem_chunks)
        if tail:
            toff = pl.multiple_of(base + full * CH, 8)
            pltpu.sync_copy(dst_hbm.at[pl.ds(toff, tail)], idx_t)
            pltpu.sync_copy(vals_hbm.at[pl.ds(toff, tail)], rows_t)
            pltpu.sync_copy(rows_t, acc_sh.at[idx_t], add=True)

        plsc.subcore_barrier()
        r0 = sid * rows_per_sub
        pltpu.sync_copy(acc_sh.at[pl.ds(r0, rows_per_sub)],
                        out_hbm.at[cid_ax, pl.ds(r0, rows_per_sub)])

    return scatter_k


def _msg_body(xjp_ref, eap_ref, w1p_ref, b1p_ref, e1_ref, e2_ref, m2_ref,
              b2p_ref, o_ref):
    xjp = xjp_ref[...]
    hidp = jnp.maximum(eap_ref[...] @ w1p_ref[...] + b1p_ref[...], 0.0)
    xb = xjp.astype(jnp.bfloat16)
    hb = hidp.astype(jnp.bfloat16)
    acc = xjp @ b2p_ref[...]
    for g in range(4):
        a = lax.dot(xb, e1_ref[g], preferred_element_type=jnp.float32)
        b = lax.dot(hb, e2_ref[g], preferred_element_type=jnp.float32)
        z = (a * b).astype(jnp.bfloat16)
        acc = acc + lax.dot(z, m2_ref[g], preferred_element_type=jnp.float32)
    o_ref[...] = acc


def _make_msg(e, h, ed, ebp):
    hp = 4 * h
    grid = (e // 4) // ebp
    full = lambda i: (0, 0)
    full3 = lambda i: (0, 0, 0)
    return pl.pallas_call(
        _msg_body,
        grid=(grid,),
        in_specs=[
            pl.BlockSpec((ebp, hp), lambda i: (i, 0)),
            pl.BlockSpec((ebp, 4 * ed), lambda i: (i, 0)),
            pl.BlockSpec((4 * ed, hp), full),
            pl.BlockSpec((1, hp), full),
            pl.BlockSpec((4, hp, h * h), full3),
            pl.BlockSpec((4, hp, h * h), full3),
            pl.BlockSpec((4, h * h, hp), full3),
            pl.BlockSpec((hp, hp), full),
        ],
        out_specs=pl.BlockSpec((ebp, hp), lambda i: (i, 0)),
        out_shape=jax.ShapeDtypeStruct((e // 4, hp), jnp.float32),
    )


def _gru_body(p0_ref, p1_ref, c0_ref, c1_ref, x_ref, root_ref, bias_ref,
              wr_ref, wz_ref, wn_ref, ur_ref, uz_ref, un_ref,
              bir_ref, biz_ref, bin_ref, bhr_ref, bhz_ref, bhn_ref, o_ref):
    x = x_ref[...]
    cnt = c0_ref[...] + c1_ref[...]
    denom = jnp.maximum(cnt, 1.0)
    agg = (p0_ref[...] + p1_ref[...]) / denom
    conv = agg + x @ root_ref[...] + bias_ref[...]
    m = jnp.maximum(conv, 0.0)
    r = jax.nn.sigmoid(m @ wr_ref[...] + bir_ref[...]
                       + x @ ur_ref[...] + bhr_ref[...])
    z = jax.nn.sigmoid(m @ wz_ref[...] + biz_ref[...]
                       + x @ uz_ref[...] + bhz_ref[...])
    nwe = jnp.tanh(m @ wn_ref[...] + bin_ref[...]
                   + r * (x @ un_ref[...] + bhn_ref[...]))
    o_ref[...] = (1.0 - z) * nwe + z * x


def _make_gru(n, h):
    np_, hp = n // 4, 4 * h
    specs = ([pl.BlockSpec((np_, hp))] * 4
             + [pl.BlockSpec((np_, hp))]
             + [pl.BlockSpec((hp, hp)), pl.BlockSpec((1, hp))]
             + [pl.BlockSpec((hp, hp))] * 6
             + [pl.BlockSpec((1, hp))] * 6)
    return pl.pallas_call(
        _gru_body,
        in_specs=specs,
        out_specs=pl.BlockSpec((np_, hp)),
        out_shape=jax.ShapeDtypeStruct((np_, hp), jnp.float32),
    )


def kernel(out, edge_index, edge_attr, W1, b1, W2, b2, root, bias,
           w_ih, w_hh, b_ih, b_hh):
    n, h = out.shape
    e, ed = edge_attr.shape
    src = edge_index[0]
    dst = edge_index[1]

    np_, ep, hp = n // 4, e // 4, 4 * h

    def bd4(w):
        return jnp.kron(jnp.eye(4, dtype=w.dtype), w)

    def tile4(v):
        return jnp.tile(v.reshape(1, h), (1, 4))

    w1p = bd4(W1.T)
    b1p = tile4(b1)
    w2r3 = W2.reshape(h, h, h)
    m2 = w2r3.transpose(0, 2, 1).reshape(h * h, h)
    b2p = bd4(b2.reshape(h, h))
    eye = jnp.eye(h, dtype=jnp.bfloat16)
    e1 = jnp.kron(eye, jnp.ones((1, h), jnp.bfloat16))
    e2 = jnp.kron(jnp.ones((1, h), jnp.bfloat16), eye)
    e1p = jnp.stack([jnp.concatenate(
        [e1 if gg == g else jnp.zeros_like(e1) for gg in range(4)], 0)
        for g in range(4)])
    e2p = jnp.stack([jnp.concatenate(
        [e2 if gg == g else jnp.zeros_like(e2) for gg in range(4)], 0)
        for g in range(4)])
    m2b = m2.astype(jnp.bfloat16)
    m2p = jnp.stack([jnp.pad(m2b, ((0, 0), (g * h, (3 - g) * h)))
                     for g in range(4)])
    wr, wz, wn = (bd4(w_ih[0:h].T), bd4(w_ih[h:2 * h].T),
                  bd4(w_ih[2 * h:3 * h].T))
    ur, uz, un = (bd4(w_hh[0:h].T), bd4(w_hh[h:2 * h].T),
                  bd4(w_hh[2 * h:3 * h].T))
    bir, biz, bin_ = (tile4(b_ih[0:h]), tile4(b_ih[h:2 * h]),
                      tile4(b_ih[2 * h:3 * h]))
    bhr, bhz, bhn = (tile4(b_hh[0:h]), tile4(b_hh[h:2 * h]),
                     tile4(b_hh[2 * h:3 * h]))
    rootp = bd4(root)
    biasp = tile4(bias)
    zeros = jnp.zeros((n, h), jnp.float32)
    ones = jnp.ones((e, h), jnp.float32)
    eap = edge_attr.reshape(ep, 4 * ed)

    gather_fn = _make_gather(n, e, h)
    scatter_fn = _make_scatter(n, e, h)
    msg_fn = _make_msg(e, h, ed, ebp=400)
    gru_fn = _make_gru(n, h)

    cntp = scatter_fn(ones, dst, zeros)
    c0 = cntp[0].reshape(np_, hp)
    c1 = cntp[1].reshape(np_, hp)
    xp = out.reshape(np_, hp)
    for _ in range(3):
        x_j = gather_fn(xp.reshape(n, h), src)
        msgp = msg_fn(x_j.reshape(ep, hp), eap, w1p, b1p, e1p, e2p, m2p, b2p)
        aggp = scatter_fn(msgp.reshape(e, h), dst, zeros)
        xp = gru_fn(aggp[0].reshape(np_, hp), aggp[1].reshape(np_, hp),
                    c0, c1, xp, rootp, biasp,
                    wr, wz, wn, ur, uz, un, bir, biz, bin_, bhr, bhz, bhn)
    return xp.reshape(n, h)


# speedup vs baseline: 4.0699x; 1.0407x over previous
import functools

import jax
import jax.numpy as jnp
from jax import lax
from jax.experimental import pallas as pl
from jax.experimental.pallas import tpu as pltpu
from jax.experimental.pallas import tpu_sc as plsc

CH = 128
K = 5


def _make_gather(n, e, h):
    mesh = plsc.VectorSubcoreMesh(core_axis_name="c", subcore_axis_name="s")
    nw = mesh.num_cores * mesh.num_subcores
    ew = e // nw
    full = ew // CH
    tail = ew - full * CH
    supers = full // K
    rem_chunks = full - supers * K

    @functools.partial(
        pl.kernel,
        out_type=jax.ShapeDtypeStruct((e, h), jnp.float32),
        mesh=mesh,
        scratch_types=[
            [pltpu.VMEM((CH,), jnp.int32) for _ in range(K)],
            [pltpu.VMEM((CH, h), jnp.float32) for _ in range(K)],
            pltpu.VMEM((8,), jnp.int32),
            pltpu.VMEM((8, h), jnp.float32),
            pltpu.SemaphoreType.DMA,
            pltpu.SemaphoreType.DMA,
            pltpu.SemaphoreType.DMA,
        ],
        compiler_params=pltpu.CompilerParams(use_tc_tiling_on_sc=False),
    )
    def gather_k(x_hbm, ei_hbm, out_hbm, idx_vs, rows_vs, idx_t, rows_t,
                 sem_i, sem_g, sem_s):
        wid = lax.axis_index("s") * mesh.num_cores + lax.axis_index("c")
        base = pl.multiple_of(wid * ew, 8)

        def run_group(first_cid, cnt):
            di = [pltpu.async_copy(
                      ei_hbm.at[pl.ds(base + (first_cid + j) * CH, CH)],
                      idx_vs[j], sem_i)
                  for j in range(cnt)]
            dg = []
            for j in range(cnt):
                di[j].wait()
                dg.append(pltpu.async_copy(x_hbm.at[idx_vs[j]], rows_vs[j],
                                           sem_g))
            ds_ = []
            for j in range(cnt):
                dg[j].wait()
                ds_.append(pltpu.async_copy(
                    rows_vs[j],
                    out_hbm.at[pl.ds(base + (first_cid + j) * CH, CH)],
                    sem_s))
            for d in ds_:
                d.wait()

        def body(g, carry):
            run_group(g * K, K)
            return carry

        lax.fori_loop(0, supers, body, 0)
        if rem_chunks:
            run_group(supers * K, rem_chunks)
        if tail:
            toff = pl.multiple_of(base + full * CH, 8)
            pltpu.sync_copy(ei_hbm.at[pl.ds(toff, tail)], idx_t)
            pltpu.async_copy(x_hbm.at[idx_t], rows_t, sem_g).wait()
            pltpu.sync_copy(rows_t, out_hbm.at[pl.ds(toff, tail)])

    return gather_k


def _make_scatter(n, e, h):
    mesh = plsc.VectorSubcoreMesh(core_axis_name="c", subcore_axis_name="s")
    nc, ns = mesh.num_cores, mesh.num_subcores
    nw = nc * ns
    ew = e // nw
    full = ew // CH
    tail = ew - full * CH
    supers = full // K
    rem_chunks = full - supers * K
    rows_per_sub = n // ns

    @functools.partial(
        pl.kernel,
        out_type=jax.ShapeDtypeStruct((nc, n, h), jnp.float32),
        mesh=mesh,
        scratch_types=[
            [pltpu.VMEM((CH,), jnp.int32) for _ in range(K)],
            [pltpu.VMEM((CH, h), jnp.float32) for _ in range(K)],
            pltpu.VMEM((8,), jnp.int32),
            pltpu.VMEM((8, h), jnp.float32),
            pltpu.VMEM_SHARED((n, h), jnp.float32),
            pltpu.SemaphoreType.DMA,
            pltpu.SemaphoreType.DMA,
            pltpu.SemaphoreType.DMA,
        ],
        compiler_params=pltpu.CompilerParams(use_tc_tiling_on_sc=False),
    )
    def scatter_k(vals_hbm, ei_hbm, zeros_hbm, out_hbm, idx_vs, rows_vs,
                  idx_t, rows_t, acc_sh, sem_i, sem_v, sem_a):
        cid_ax = lax.axis_index("c")
        sid = lax.axis_index("s")
        wid = sid * nc + cid_ax
        base = pl.multiple_of(wid * ew, 8)

        @pl.when(sid == 0)
        def _():
            pltpu.sync_copy(zeros_hbm, acc_sh)

        plsc.subcore_barrier()

        def run_group(first_cid, cnt):
            di, dv = [], []
            for j in range(cnt):
                off = base + (first_cid + j) * CH
                di.append(pltpu.async_copy(ei_hbm.at[pl.ds(e + off, CH)],
                                           idx_vs[j], sem_i))
                dv.append(pltpu.async_copy(vals_hbm.at[pl.ds(off, CH)],
                                           rows_vs[j], sem_v))
            da = []
            for j in range(cnt):
                di[j].wait()
                dv[j].wait()
                da.append(pltpu.async_copy(rows_vs[j], acc_sh.at[idx_vs[j]],
                                           sem_a, add=True))
            for d in da:
                d.wait()

        def body(g, carry):
            run_group(g * K, K)
            return carry

        lax.fori_loop(0, supers, body, 0)
        if rem_chunks:
            run_group(supers * K, rem_chunks)
        if tail:
            toff = pl.multiple_of(base + full * CH, 8)
            pltpu.sync_copy(ei_hbm.at[pl.ds(e + toff, tail)], idx_t)
            pltpu.sync_copy(vals_hbm.at[pl.ds(toff, tail)], rows_t)
            pltpu.sync_copy(rows_t, acc_sh.at[idx_t], add=True)

        plsc.subcore_barrier()
        r0 = sid * rows_per_sub
        pltpu.sync_copy(acc_sh.at[pl.ds(r0, rows_per_sub)],
                        out_hbm.at[cid_ax, pl.ds(r0, rows_per_sub)])

    return scatter_k


def _msg_body(xjp_ref, eap_ref, w1p_ref, b1p_ref, e1_ref, e2_ref, m2_ref,
              b2p_ref, o_ref):
    xjp = xjp_ref[...]
    hidp = jnp.maximum(eap_ref[...] @ w1p_ref[...] + b1p_ref[...], 0.0)
    xb = xjp.astype(jnp.bfloat16)
    hb = hidp.astype(jnp.bfloat16)
    acc = xjp @ b2p_ref[...]
    for g in range(4):
        a = lax.dot(xb, e1_ref[g], preferred_element_type=jnp.float32)
        b = lax.dot(hb, e2_ref[g], preferred_element_type=jnp.float32)
        z = (a * b).astype(jnp.bfloat16)
        acc = acc + lax.dot(z, m2_ref[g], preferred_element_type=jnp.float32)
    o_ref[...] = acc


def _make_msg(e, h, ed, ebp):
    hp = 4 * h
    grid = (e // 4) // ebp
    full = lambda i: (0, 0)
    full3 = lambda i: (0, 0, 0)
    return pl.pallas_call(
        _msg_body,
        grid=(grid,),
        in_specs=[
            pl.BlockSpec((ebp, hp), lambda i: (i, 0)),
            pl.BlockSpec((ebp, 4 * ed), lambda i: (i, 0)),
            pl.BlockSpec((4 * ed, hp), full),
            pl.BlockSpec((1, hp), full),
            pl.BlockSpec((4, hp, h * h), full3),
            pl.BlockSpec((4, hp, h * h), full3),
            pl.BlockSpec((4, h * h, hp), full3),
            pl.BlockSpec((hp, hp), full),
        ],
        out_specs=pl.BlockSpec((ebp, hp), lambda i: (i, 0)),
        out_shape=jax.ShapeDtypeStruct((e // 4, hp), jnp.float32),
    )


def _gru_body(p0_ref, p1_ref, c0_ref, c1_ref, x_ref, root_ref, bias_ref,
              wr_ref, wz_ref, wn_ref, ur_ref, uz_ref, un_ref,
              bir_ref, biz_ref, bin_ref, bhr_ref, bhz_ref, bhn_ref, o_ref):
    x = x_ref[...]
    cnt = c0_ref[...] + c1_ref[...]
    denom = jnp.maximum(cnt, 1.0)
    agg = (p0_ref[...] + p1_ref[...]) / denom
    conv = agg + x @ root_ref[...] + bias_ref[...]
    m = jnp.maximum(conv, 0.0)
    r = jax.nn.sigmoid(m @ wr_ref[...] + bir_ref[...]
                       + x @ ur_ref[...] + bhr_ref[...])
    z = jax.nn.sigmoid(m @ wz_ref[...] + biz_ref[...]
                       + x @ uz_ref[...] + bhz_ref[...])
    nwe = jnp.tanh(m @ wn_ref[...] + bin_ref[...]
                   + r * (x @ un_ref[...] + bhn_ref[...]))
    o_ref[...] = (1.0 - z) * nwe + z * x


def _make_gru(n, h):
    np_, hp = n // 4, 4 * h
    specs = ([pl.BlockSpec((np_, hp))] * 4
             + [pl.BlockSpec((np_, hp))]
             + [pl.BlockSpec((hp, hp)), pl.BlockSpec((1, hp))]
             + [pl.BlockSpec((hp, hp))] * 6
             + [pl.BlockSpec((1, hp))] * 6)
    return pl.pallas_call(
        _gru_body,
        in_specs=specs,
        out_specs=pl.BlockSpec((np_, hp)),
        out_shape=jax.ShapeDtypeStruct((np_, hp), jnp.float32),
    )


def kernel(out, edge_index, edge_attr, W1, b1, W2, b2, root, bias,
           w_ih, w_hh, b_ih, b_hh):
    n, h = out.shape
    e, ed = edge_attr.shape
    ei1d = edge_index.reshape(2 * e)

    np_, ep, hp = n // 4, e // 4, 4 * h

    def bd4(w):
        return jnp.kron(jnp.eye(4, dtype=w.dtype), w)

    def tile4(v):
        return jnp.tile(v.reshape(1, h), (1, 4))

    w1p = bd4(W1.T)
    b1p = tile4(b1)
    w2r3 = W2.reshape(h, h, h)
    m2 = w2r3.transpose(0, 2, 1).reshape(h * h, h)
    b2p = bd4(b2.reshape(h, h))
    eye = jnp.eye(h, dtype=jnp.bfloat16)
    e1 = jnp.kron(eye, jnp.ones((1, h), jnp.bfloat16))
    e2 = jnp.kron(jnp.ones((1, h), jnp.bfloat16), eye)
    e1p = jnp.stack([jnp.concatenate(
        [e1 if gg == g else jnp.zeros_like(e1) for gg in range(4)], 0)
        for g in range(4)])
    e2p = jnp.stack([jnp.concatenate(
        [e2 if gg == g else jnp.zeros_like(e2) for gg in range(4)], 0)
        for g in range(4)])
    m2b = m2.astype(jnp.bfloat16)
    m2p = jnp.stack([jnp.pad(m2b, ((0, 0), (g * h, (3 - g) * h)))
                     for g in range(4)])
    wr, wz, wn = (bd4(w_ih[0:h].T), bd4(w_ih[h:2 * h].T),
                  bd4(w_ih[2 * h:3 * h].T))
    ur, uz, un = (bd4(w_hh[0:h].T), bd4(w_hh[h:2 * h].T),
                  bd4(w_hh[2 * h:3 * h].T))
    bir, biz, bin_ = (tile4(b_ih[0:h]), tile4(b_ih[h:2 * h]),
                      tile4(b_ih[2 * h:3 * h]))
    bhr, bhz, bhn = (tile4(b_hh[0:h]), tile4(b_hh[h:2 * h]),
                     tile4(b_hh[2 * h:3 * h]))
    rootp = bd4(root)
    biasp = tile4(bias)
    zeros = jnp.zeros((n, h), jnp.float32)
    ones = jnp.ones((e, h), jnp.float32)
    eap = edge_attr.reshape(ep, 4 * ed)

    gather_fn = _make_gather(n, e, h)
    scatter_fn = _make_scatter(n, e, h)
    msg_fn = _make_msg(e, h, ed, ebp=800)
    gru_fn = _make_gru(n, h)

    cntp = scatter_fn(ones, ei1d, zeros)
    c0 = cntp[0].reshape(np_, hp)
    c1 = cntp[1].reshape(np_, hp)
    xp = out.reshape(np_, hp)
    for _ in range(3):
        x_j = gather_fn(xp.reshape(n, h), ei1d)
        msgp = msg_fn(x_j.reshape(ep, hp), eap, w1p, b1p, e1p, e2p, m2p, b2p)
        aggp = scatter_fn(msgp.reshape(e, h), ei1d, zeros)
        xp = gru_fn(aggp[0].reshape(np_, hp), aggp[1].reshape(np_, hp),
                    c0, c1, xp, rootp, biasp,
                    wr, wz, wn, ur, uz, un, bir, biz, bin_, bhr, bhz, bhn)
    return xp.reshape(n, h)
